# baseline (device time: 47547 ns/iter reference)
import jax
import jax.numpy as jnp
from jax import lax
from jax.experimental import pallas as pl
from jax.experimental.pallas import tpu as pltpu


def kernel(Q, K, V):
    B, SQ, H, D = Q.shape
    SKV = K.shape[1]
    scale = D ** -0.5

    def body(q_ref, k_ref, v_ref, o_ref, send_buf, recv_buf, send_sem, recv_sem):
        ix = lax.axis_index("x")
        iy = lax.axis_index("y")
        iz = lax.axis_index("z")
        nbr = (1 - ix, iy, iz)

        barrier = pltpu.get_barrier_semaphore()
        pl.semaphore_signal(
            barrier, inc=1, device_id=nbr, device_id_type=pl.DeviceIdType.MESH
        )

        for b in range(B):
            for h in range(H):
                q = q_ref[b, :, h, :]
                k = k_ref[b, :, h, :]
                v = v_ref[b, :, h, :]
                s = lax.dot_general(
                    q, k, (((1,), (1,)), ((), ())),
                    preferred_element_type=jnp.float32,
                ) * scale
                p = jnp.exp(s)
                l = jnp.sum(p, axis=1, keepdims=True)
                o = lax.dot_general(
                    p, v, (((1,), (0,)), ((), ())),
                    preferred_element_type=jnp.float32,
                )
                row = b * H + h
                send_buf[pl.ds(row, 1), pl.ds(0, D)] = o
                send_buf[pl.ds(row, 1), pl.ds(D, D)] = jnp.broadcast_to(l, (1, D))

        pl.semaphore_wait(barrier, 1)
        rdma = pltpu.make_async_remote_copy(
            src_ref=send_buf,
            dst_ref=recv_buf,
            send_sem=send_sem,
            recv_sem=recv_sem,
            device_id=nbr,
            device_id_type=pl.DeviceIdType.MESH,
        )
        rdma.start()
        rdma.wait()

        tot = send_buf[...] + recv_buf[...]
        o_all = tot[:, :D] / tot[:, D:D + 1]
        o_ref[...] = o_all.reshape(B, SQ, H, D)

    return pl.pallas_call(
        body,
        out_shape=jax.ShapeDtypeStruct((B, SQ, H, D), jnp.float32),
        in_specs=[pl.BlockSpec(memory_space=pltpu.VMEM)] * 3,
        out_specs=pl.BlockSpec(memory_space=pltpu.VMEM),
        scratch_shapes=[
            pltpu.VMEM((B * H, 2 * D), jnp.float32),
            pltpu.VMEM((B * H, 2 * D), jnp.float32),
            pltpu.SemaphoreType.DMA,
            pltpu.SemaphoreType.DMA,
        ],
        compiler_params=pltpu.CompilerParams(collective_id=0),
    )(Q, K, V)


# device time: 20634 ns/iter; 2.3043x vs baseline; 2.3043x over previous
import jax
import jax.numpy as jnp
from jax import lax
from jax.experimental import pallas as pl
from jax.experimental.pallas import tpu as pltpu


def kernel(Q, K, V):
    B, SQ, H, D = Q.shape
    SKV = K.shape[1]
    HD = H * D
    scale = D ** -0.5

    K2 = K.reshape(B, SKV, HD)
    V2 = V.reshape(B, SKV, HD)
    eye = jnp.eye(H, dtype=Q.dtype)
    Qblk = (Q[:, 0, :, None, :] * eye[None, :, :, None]).reshape(B, H, HD)

    def body(q_ref, k_ref, v_ref, o_ref, send_buf, recv_buf, send_sem, recv_sem):
        ix = lax.axis_index("x")
        iy = lax.axis_index("y")
        iz = lax.axis_index("z")
        nbr = (1 - ix, iy, iz)

        barrier = pltpu.get_barrier_semaphore()
        pl.semaphore_signal(
            barrier, inc=1, device_id=nbr, device_id_type=pl.DeviceIdType.MESH
        )

        for b in range(B):
            qb = q_ref[b].astype(jnp.bfloat16)
            kb = k_ref[b].astype(jnp.bfloat16)
            vb = v_ref[b].astype(jnp.bfloat16)
            s = lax.dot_general(
                qb, kb, (((1,), (1,)), ((), ())),
                preferred_element_type=jnp.float32,
            ) * scale
            p = jnp.exp(s)
            l = jnp.sum(p, axis=1, keepdims=True)
            r = lax.dot_general(
                p.astype(jnp.bfloat16), vb, (((1,), (0,)), ((), ())),
                preferred_element_type=jnp.float32,
            )
            for h in range(H):
                send_buf[pl.ds(b * H + h, 1), pl.ds(0, D)] = (
                    r[h:h + 1, h * D:(h + 1) * D]
                )
            send_buf[pl.ds(b * H, H), pl.ds(D, D)] = jnp.broadcast_to(l, (H, D))

        pl.semaphore_wait(barrier, 1)
        rdma = pltpu.make_async_remote_copy(
            src_ref=send_buf,
            dst_ref=recv_buf,
            send_sem=send_sem,
            recv_sem=recv_sem,
            device_id=nbr,
            device_id_type=pl.DeviceIdType.MESH,
        )
        rdma.start()
        rdma.wait()

        tot = send_buf[...] + recv_buf[...]
        o_all = tot[:, :D] / tot[:, D:D + 1]
        o_ref[...] = o_all.reshape(B, SQ, H, D)

    return pl.pallas_call(
        body,
        out_shape=jax.ShapeDtypeStruct((B, SQ, H, D), jnp.float32),
        in_specs=[pl.BlockSpec(memory_space=pltpu.VMEM)] * 3,
        out_specs=pl.BlockSpec(memory_space=pltpu.VMEM),
        scratch_shapes=[
            pltpu.VMEM((B * H, 2 * D), jnp.float32),
            pltpu.VMEM((B * H, 2 * D), jnp.float32),
            pltpu.SemaphoreType.DMA,
            pltpu.SemaphoreType.DMA,
        ],
        compiler_params=pltpu.CompilerParams(collective_id=0),
    )(Qblk, K2, V2)
